# baseline (speedup 1.0000x reference)
import functools

import jax
import jax.numpy as jnp
from jax import lax
from jax.experimental import pallas as pl
from jax.experimental.pallas import tpu as pltpu
from jax.experimental.pallas import tpu_sc as plsc

N = 800000
D = 64
NP = N // 2
T2_ROWS = 3600

NC = 2
NS = 16
NW = NC * NS
C = 320
NCHT = NP // C
ITERS = -(-NCHT // NW)
GP = C // 16
SPLITS = ((0, 128), (128, 128), (256, 64))

_mesh = plsc.VectorSubcoreMesh(core_axis_name="c", subcore_axis_name="s")


@functools.partial(
    pl.kernel,
    out_type=jax.ShapeDtypeStruct((NP, 2 * D), jnp.float32),
    mesh=_mesh,
    scratch_types=[
        pltpu.VMEM((6, C), jnp.int32),
        pltpu.VMEM((6, C), jnp.int32),
        pltpu.VMEM((C,), jnp.int32),
        pltpu.VMEM((C,), jnp.int32),
        pltpu.VMEM((C, 2 * D), jnp.float32),
        pltpu.VMEM((C, 2 * D), jnp.float32),
        pltpu.SemaphoreType.DMA,
        pltpu.SemaphoreType.DMA,
        pltpu.SemaphoreType.DMA,
        pltpu.SemaphoreType.DMA,
        pltpu.SemaphoreType.DMA,
        pltpu.SemaphoreType.DMA,
    ],
)
def _bond_encode(ea_hbm, t2_hbm, out_hbm,
                 attr0, attr1, idx0, idx1, rows0, rows1,
                 sa0, sa1, sg0, sg1, so0, so1):
    wid = lax.axis_index("s") * NC + lax.axis_index("c")
    attr = (attr0, attr1)
    idx = (idx0, idx1)
    rows = (rows0, rows1)
    sa = (sa0, sa1)
    sg = (sg0, sg1)
    so = (so0, so1)

    def cid_of(i):
        return wid + i * NW

    def start_attr(i, b):
        pltpu.async_copy(ea_hbm.at[cid_of(i)], attr[b], sa[b])

    def wait_attr(b):
        pltpu.make_async_copy(ea_hbm.at[0], attr[b], sa[b]).wait()

    def compute_idx(b):
        a = attr[b]
        v = idx[b]
        for g in range(GP):
            s = pl.ds(g * 16, 16)
            ie = a[0, s] * 12 + a[1, s] * 2 + a[2, s]
            io = a[3, s] * 12 + a[4, s] * 2 + a[5, s]
            v[s] = ie * 60 + io

    def start_gather(b):
        for off, cnt in SPLITS:
            pltpu.async_copy(
                t2_hbm.at[idx[b].at[pl.ds(off, cnt)]],
                rows[b].at[pl.ds(off, cnt)],
                sg[b],
            )

    def wait_gather(b):
        for off, cnt in SPLITS:
            pltpu.make_async_copy(
                t2_hbm.at[pl.ds(0, cnt)],
                rows[b].at[pl.ds(off, cnt)],
                sg[b],
            ).wait()

    def start_out(i, b):
        pltpu.async_copy(rows[b], out_hbm.at[pl.ds(cid_of(i) * C, C)], so[b])

    def wait_out(b):
        pltpu.make_async_copy(rows[b], out_hbm.at[pl.ds(0, C)], so[b]).wait()

    start_attr(0, 0)
    start_attr(1, 1)

    def super_body(sv):
        for b in range(2):
            i = sv * 2 + b

            @pl.when((i < ITERS) & (cid_of(i) < NCHT))
            def _():
                wait_attr(b)
                compute_idx(b)

                @pl.when(i >= 2)
                def _():
                    wait_out(b)

                start_gather(b)

                @pl.when((i + 2 < ITERS) & (cid_of(i + 2) < NCHT))
                def _():
                    start_attr(i + 2, b)

            @pl.when((i >= 1) & (cid_of(i - 1) < NCHT))
            def _():
                wait_gather(1 - b)
                start_out(i - 1, 1 - b)

    pl.loop(0, (ITERS + 2) // 2)(super_body)

    for j in (ITERS - 2, ITERS - 1):
        @pl.when(cid_of(j) < NCHT)
        def _():
            wait_out(j % 2)


RP_B = 800


def _repack_body(x_ref, o_ref):
    x = x_ref[...]
    o_ref[...] = jnp.stack([x[:, :D], x[:, D:]], axis=1).reshape(2 * RP_B, D)


_repack = pl.pallas_call(
    _repack_body,
    grid=(NP // RP_B,),
    in_specs=[pl.BlockSpec((RP_B, 2 * D), lambda g: (g, 0))],
    out_specs=pl.BlockSpec((2 * RP_B, D), lambda g: (g, 0)),
    out_shape=jax.ShapeDtypeStruct((N, D), jnp.float32),
)


def kernel(edge_attr, W0, W1, W2):
    t = (W0[:, None, None, :] + W1[None, :, None, :] + W2[None, None, :, :])
    t = t.reshape(60, D)
    t2 = jnp.concatenate(
        [jnp.broadcast_to(t[:, None, :], (60, 60, D)),
         jnp.broadcast_to(t[None, :, :], (60, 60, D))], axis=-1,
    ).reshape(T2_ROWS, 2 * D)
    ea = edge_attr.reshape(NCHT, C, 2, 3)
    ea_cm = ea.transpose(0, 2, 3, 1).reshape(NCHT, 6, C)
    out2 = _bond_encode(ea_cm, t2)
    return _repack(out2)


# speedup vs baseline: 1.3663x; 1.3663x over previous
import functools

import jax
import jax.numpy as jnp
from jax import lax
from jax.experimental import pallas as pl
from jax.experimental.pallas import tpu as pltpu
from jax.experimental.pallas import tpu_sc as plsc

N = 800000
D = 64
NP = N // 2
T2_ROWS = 3600

NC = 2
NS = 16
NW = NC * NS
C = 320
NCHT = NP // C
ITERS = -(-NCHT // NW)
GP = C // 16
SPLITS = ((0, 128), (128, 128), (256, 64))


IDX_B = 8192
IDX_PAD = 98 * IDX_B


def _idx_body(x_ref, o_ref):
    x = x_ref[...]
    o_ref[...] = x[:, 0] * 12 + x[:, 1] * 2 + x[:, 2]


_fuse_idx = pl.pallas_call(
    _idx_body,
    grid=(IDX_PAD // IDX_B,),
    in_specs=[pl.BlockSpec((IDX_B, 3), lambda g: (g, 0))],
    out_specs=pl.BlockSpec((IDX_B,), lambda g: (g,)),
    out_shape=jax.ShapeDtypeStruct((IDX_PAD,), jnp.int32),
)


_mesh = plsc.VectorSubcoreMesh(core_axis_name="c", subcore_axis_name="s")


@functools.partial(
    pl.kernel,
    out_type=jax.ShapeDtypeStruct((NP, 2 * D), jnp.float32),
    mesh=_mesh,
    scratch_types=[
        pltpu.VMEM((C,), jnp.int32),
        pltpu.VMEM((C,), jnp.int32),
        pltpu.VMEM((C,), jnp.int32),
        pltpu.VMEM((C,), jnp.int32),
        pltpu.VMEM((C,), jnp.int32),
        pltpu.VMEM((C,), jnp.int32),
        pltpu.VMEM((C, 2 * D), jnp.float32),
        pltpu.VMEM((C, 2 * D), jnp.float32),
        pltpu.SemaphoreType.DMA,
        pltpu.SemaphoreType.DMA,
        pltpu.SemaphoreType.DMA,
        pltpu.SemaphoreType.DMA,
        pltpu.SemaphoreType.DMA,
        pltpu.SemaphoreType.DMA,
    ],
)
def _bond_encode(idx_hbm, t2_hbm, out_hbm,
                 ia0, ia1, ib0, ib1, kk0, kk1, rows0, rows1,
                 sa0, sa1, sg0, sg1, so0, so1):
    wid = lax.axis_index("s") * NC + lax.axis_index("c")
    ia = (ia0, ia1)
    ib = (ib0, ib1)
    keys = (kk0, kk1)
    rows = (rows0, rows1)
    sa = (sa0, sa1)
    sg = (sg0, sg1)
    so = (so0, so1)

    def cid_of(i):
        return wid + i * NW

    def start_idx(i, b):
        r0 = cid_of(i) * C
        pltpu.async_copy(idx_hbm.at[pl.ds(r0, C)], ia[b], sa[b])
        pltpu.async_copy(idx_hbm.at[pl.ds(NP + r0, C)], ib[b], sa[b])

    def wait_idx(b):
        pltpu.make_async_copy(idx_hbm.at[pl.ds(0, C)], ia[b], sa[b]).wait()
        pltpu.make_async_copy(idx_hbm.at[pl.ds(0, C)], ib[b], sa[b]).wait()

    def fuse_keys(b):
        for g in range(GP):
            s = pl.ds(g * 16, 16)
            keys[b][s] = ia[b][s] * 60 + ib[b][s]

    def start_gather(b):
        for off, cnt in SPLITS:
            pltpu.async_copy(
                t2_hbm.at[keys[b].at[pl.ds(off, cnt)]],
                rows[b].at[pl.ds(off, cnt)],
                sg[b],
            )

    def wait_gather(b):
        for off, cnt in SPLITS:
            pltpu.make_async_copy(
                t2_hbm.at[pl.ds(0, cnt)],
                rows[b].at[pl.ds(off, cnt)],
                sg[b],
            ).wait()

    def start_out(i, b):
        pltpu.async_copy(rows[b], out_hbm.at[pl.ds(cid_of(i) * C, C)], so[b])

    def wait_out(b):
        pltpu.make_async_copy(rows[b], out_hbm.at[pl.ds(0, C)], so[b]).wait()

    start_idx(0, 0)
    start_idx(1, 1)

    def super_body(sv):
        for b in range(2):
            i = sv * 2 + b

            @pl.when((i < ITERS) & (cid_of(i) < NCHT))
            def _():
                wait_idx(b)
                fuse_keys(b)

                @pl.when(i >= 2)
                def _():
                    wait_out(b)

                start_gather(b)

                @pl.when((i + 2 < ITERS) & (cid_of(i + 2) < NCHT))
                def _():
                    start_idx(i + 2, b)

            @pl.when((i >= 1) & (cid_of(i - 1) < NCHT))
            def _():
                wait_gather(1 - b)
                start_out(i - 1, 1 - b)

    pl.loop(0, (ITERS + 2) // 2)(super_body)

    for j in (ITERS - 2, ITERS - 1):
        @pl.when(cid_of(j) < NCHT)
        def _():
            wait_out(j % 2)


RP_B = 3200
RP_HALF = NP // RP_B


def _repack_body(x_ref, o_ref):
    h = pl.program_id(0)
    x = x_ref[...]
    o_ref[...] = jnp.where(h < RP_HALF, x[:, :D], x[:, D:])


_repack = pl.pallas_call(
    _repack_body,
    grid=(2 * RP_HALF,),
    in_specs=[pl.BlockSpec((RP_B, 2 * D), lambda h: (h % RP_HALF, 0))],
    out_specs=pl.BlockSpec((RP_B, D), lambda h: (h, 0)),
    out_shape=jax.ShapeDtypeStruct((N, D), jnp.float32),
)


def kernel(edge_attr, W0, W1, W2):
    t = (W0[:, None, None, :] + W1[None, :, None, :] + W2[None, None, :, :])
    t = t.reshape(60, D)
    t2 = jnp.concatenate(
        [jnp.broadcast_to(t[:, None, :], (60, 60, D)),
         jnp.broadcast_to(t[None, :, :], (60, 60, D))], axis=-1,
    ).reshape(T2_ROWS, 2 * D)
    idx_e = _fuse_idx(edge_attr)
    out2 = _bond_encode(idx_e, t2)
    return _repack(out2)


# speedup vs baseline: 2.7550x; 2.0164x over previous
import functools

import jax
import jax.numpy as jnp
from jax import lax
from jax.experimental import pallas as pl
from jax.experimental.pallas import tpu as pltpu
from jax.experimental.pallas import tpu_sc as plsc

N = 800000
D = 64
NP = N // 2
T2_ROWS = 3600

NC = 2
NS = 16
NW = NC * NS
C = 320
NCHT = NP // C
ITERS = -(-NCHT // NW)
GP = C // 16
SPLITS = ((0, 128), (128, 128), (256, 64))

_mesh = plsc.VectorSubcoreMesh(core_axis_name="c", subcore_axis_name="s")


@functools.partial(
    pl.kernel,
    out_type=jax.ShapeDtypeStruct((NP, 2 * D), jnp.float32),
    mesh=_mesh,
    scratch_types=[
        [pltpu.VMEM((C,), jnp.int32) for _ in range(6)],
        [pltpu.VMEM((C,), jnp.int32) for _ in range(6)],
        pltpu.VMEM((C,), jnp.int32),
        pltpu.VMEM((C,), jnp.int32),
        pltpu.VMEM((C, 2 * D), jnp.float32),
        pltpu.VMEM((C, 2 * D), jnp.float32),
        pltpu.SemaphoreType.DMA,
        pltpu.SemaphoreType.DMA,
        pltpu.SemaphoreType.DMA,
        pltpu.SemaphoreType.DMA,
        pltpu.SemaphoreType.DMA,
        pltpu.SemaphoreType.DMA,
    ],
)
def _bond_encode(ea0_hbm, ea1_hbm, ea2_hbm, t2_hbm, out_hbm,
                 attr0, attr1, kk0, kk1, rows0, rows1,
                 sa0, sa1, sg0, sg1, so0, so1):
    ea_cols = (ea0_hbm, ea1_hbm, ea2_hbm)
    wid = lax.axis_index("s") * NC + lax.axis_index("c")
    attr = (attr0, attr1)
    keys = (kk0, kk1)
    rows = (rows0, rows1)
    sa = (sa0, sa1)
    sg = (sg0, sg1)
    so = (so0, so1)

    def cid_of(i):
        return wid + i * NW

    def start_attr(i, b):
        r0 = cid_of(i) * C
        for k in range(3):
            pltpu.async_copy(ea_cols[k].at[pl.ds(r0, C)],
                             attr[b][k], sa[b])
            pltpu.async_copy(ea_cols[k].at[pl.ds(NP + r0, C)],
                             attr[b][3 + k], sa[b])

    def wait_attr(b):
        for k in range(6):
            pltpu.make_async_copy(ea0_hbm.at[pl.ds(0, C)],
                                  attr[b][k], sa[b]).wait()

    def fuse_keys(b):
        a = attr[b]
        v = keys[b]
        for g in range(GP):
            s = pl.ds(g * 16, 16)
            ie = a[0][s] * 12 + a[1][s] * 2 + a[2][s]
            io = a[3][s] * 12 + a[4][s] * 2 + a[5][s]
            v[s] = ie * 60 + io

    def start_gather(b):
        for off, cnt in SPLITS:
            pltpu.async_copy(
                t2_hbm.at[keys[b].at[pl.ds(off, cnt)]],
                rows[b].at[pl.ds(off, cnt)],
                sg[b],
            )

    def wait_gather(b):
        for off, cnt in SPLITS:
            pltpu.make_async_copy(
                t2_hbm.at[pl.ds(0, cnt)],
                rows[b].at[pl.ds(off, cnt)],
                sg[b],
            ).wait()

    def start_out(i, b):
        pltpu.async_copy(rows[b], out_hbm.at[pl.ds(cid_of(i) * C, C)], so[b])

    def wait_out(b):
        pltpu.make_async_copy(rows[b], out_hbm.at[pl.ds(0, C)], so[b]).wait()

    start_attr(0, 0)
    start_attr(1, 1)

    def super_body(sv):
        for b in range(2):
            i = sv * 2 + b

            @pl.when((i < ITERS) & (cid_of(i) < NCHT))
            def _():
                wait_attr(b)
                fuse_keys(b)

                @pl.when(i >= 2)
                def _():
                    wait_out(b)

                start_gather(b)

                @pl.when((i + 2 < ITERS) & (cid_of(i + 2) < NCHT))
                def _():
                    start_attr(i + 2, b)

            @pl.when((i >= 1) & (cid_of(i - 1) < NCHT))
            def _():
                wait_gather(1 - b)
                start_out(i - 1, 1 - b)

    pl.loop(0, (ITERS + 2) // 2)(super_body)

    for j in (ITERS - 2, ITERS - 1):
        @pl.when(cid_of(j) < NCHT)
        def _():
            wait_out(j % 2)


RP_B = 3200
RP_HALF = NP // RP_B


def _repack_body(x_ref, o_ref):
    h = pl.program_id(0)
    x = x_ref[...]
    half = jnp.where(h < RP_HALF, x[:, :D], x[:, D:])
    eye = (lax.broadcasted_iota(jnp.int32, (D, D), 0)
           == lax.broadcasted_iota(jnp.int32, (D, D), 1)).astype(jnp.float32)
    o_ref[...] = jax.lax.dot_general(
        eye, half, (((1,), (1,)), ((), ())),
        preferred_element_type=jnp.float32,
    )


_repack = pl.pallas_call(
    _repack_body,
    grid=(2 * RP_HALF,),
    in_specs=[pl.BlockSpec((RP_B, 2 * D), lambda h: (h % RP_HALF, 0))],
    out_specs=pl.BlockSpec((D, RP_B), lambda h: (0, h)),
    out_shape=jax.ShapeDtypeStruct((D, N), jnp.float32),
)


def kernel(edge_attr, W0, W1, W2):
    t = (W0[:, None, None, :] + W1[None, :, None, :] + W2[None, None, :, :])
    t = t.reshape(60, D)
    t2 = jnp.concatenate(
        [jnp.broadcast_to(t[:, None, :], (60, 60, D)),
         jnp.broadcast_to(t[None, :, :], (60, 60, D))], axis=-1,
    ).reshape(T2_ROWS, 2 * D)
    out2 = _bond_encode(edge_attr[:, 0], edge_attr[:, 1], edge_attr[:, 2], t2)
    return _repack(out2).T


# speedup vs baseline: 2.7558x; 1.0003x over previous
import functools

import jax
import jax.numpy as jnp
from jax import lax
from jax.experimental import pallas as pl
from jax.experimental.pallas import tpu as pltpu
from jax.experimental.pallas import tpu_sc as plsc

N = 800000
D = 64
NP = N // 2
T2_ROWS = 3600

NC = 2
NS = 16
NW = NC * NS
C = 128
NB = 4
NCHT = NP // C
ITERS = -(-NCHT // NW)
GP = C // 16
SPLITS = ((0, 128),)

_mesh = plsc.VectorSubcoreMesh(core_axis_name="c", subcore_axis_name="s")


@functools.partial(
    pl.kernel,
    out_type=jax.ShapeDtypeStruct((NP, 2 * D), jnp.float32),
    mesh=_mesh,
    scratch_types=[
        [[pltpu.VMEM((C,), jnp.int32) for _ in range(6)] for _ in range(NB)],
        [pltpu.VMEM((C,), jnp.int32) for _ in range(NB)],
        [pltpu.VMEM((C, 2 * D), jnp.float32) for _ in range(NB)],
        [pltpu.SemaphoreType.DMA for _ in range(NB)],
        [pltpu.SemaphoreType.DMA for _ in range(NB)],
        [pltpu.SemaphoreType.DMA for _ in range(NB)],
    ],
)
def _bond_encode(ea0_hbm, ea1_hbm, ea2_hbm, t2_hbm, out_hbm,
                 attr, keys, rows, sa, sg, so):
    ea_cols = (ea0_hbm, ea1_hbm, ea2_hbm)
    wid = lax.axis_index("s") * NC + lax.axis_index("c")

    def cid_of(i):
        return wid + i * NW

    def start_attr(i, b):
        r0 = cid_of(i) * C
        for k in range(3):
            pltpu.async_copy(ea_cols[k].at[pl.ds(r0, C)],
                             attr[b][k], sa[b])
            pltpu.async_copy(ea_cols[k].at[pl.ds(NP + r0, C)],
                             attr[b][3 + k], sa[b])

    def wait_attr(b):
        for k in range(6):
            pltpu.make_async_copy(ea0_hbm.at[pl.ds(0, C)],
                                  attr[b][k], sa[b]).wait()

    def fuse_keys(b):
        a = attr[b]
        v = keys[b]
        for g in range(GP):
            s = pl.ds(g * 16, 16)
            ie = a[0][s] * 12 + a[1][s] * 2 + a[2][s]
            io = a[3][s] * 12 + a[4][s] * 2 + a[5][s]
            v[s] = ie * 60 + io

    def start_gather(b):
        for off, cnt in SPLITS:
            pltpu.async_copy(
                t2_hbm.at[keys[b].at[pl.ds(off, cnt)]],
                rows[b].at[pl.ds(off, cnt)],
                sg[b],
            )

    def wait_gather(b):
        for off, cnt in SPLITS:
            pltpu.make_async_copy(
                t2_hbm.at[pl.ds(0, cnt)],
                rows[b].at[pl.ds(off, cnt)],
                sg[b],
            ).wait()

    def start_out(i, b):
        pltpu.async_copy(rows[b], out_hbm.at[pl.ds(cid_of(i) * C, C)], so[b])

    def wait_out(b):
        pltpu.make_async_copy(rows[b], out_hbm.at[pl.ds(0, C)], so[b]).wait()

    for b0 in range(NB):
        start_attr(b0, b0)

    def super_body(sv):
        for b in range(NB):
            i = sv * NB + b

            @pl.when((i < ITERS) & (cid_of(i) < NCHT))
            def _():
                wait_attr(b)
                fuse_keys(b)

                @pl.when(i >= NB)
                def _():
                    wait_out(b)

                start_gather(b)

                @pl.when((i + NB < ITERS) & (cid_of(i + NB) < NCHT))
                def _():
                    start_attr(i + NB, b)

            @pl.when((i >= 1) & (cid_of(i - 1) < NCHT))
            def _():
                wait_gather((b - 1) % NB)
                start_out(i - 1, (b - 1) % NB)

    pl.loop(0, (ITERS + NB) // NB)(super_body)

    for j in range(ITERS - NB, ITERS):
        @pl.when(cid_of(j) < NCHT)
        def _():
            wait_out(j % NB)


RP_B = 3200
RP_HALF = NP // RP_B


def _repack_body(x_ref, o_ref):
    h = pl.program_id(0)
    x = x_ref[...]
    half = jnp.where(h < RP_HALF, x[:, :D], x[:, D:])
    eye = (lax.broadcasted_iota(jnp.int32, (D, D), 0)
           == lax.broadcasted_iota(jnp.int32, (D, D), 1)).astype(jnp.float32)
    o_ref[...] = jax.lax.dot_general(
        eye, half, (((1,), (1,)), ((), ())),
        preferred_element_type=jnp.float32,
    )


_repack = pl.pallas_call(
    _repack_body,
    grid=(2 * RP_HALF,),
    in_specs=[pl.BlockSpec((RP_B, 2 * D), lambda h: (h % RP_HALF, 0))],
    out_specs=pl.BlockSpec((D, RP_B), lambda h: (0, h)),
    out_shape=jax.ShapeDtypeStruct((D, N), jnp.float32),
)


def kernel(edge_attr, W0, W1, W2):
    t = (W0[:, None, None, :] + W1[None, :, None, :] + W2[None, None, :, :])
    t = t.reshape(60, D)
    t2 = jnp.concatenate(
        [jnp.broadcast_to(t[:, None, :], (60, 60, D)),
         jnp.broadcast_to(t[None, :, :], (60, 60, D))], axis=-1,
    ).reshape(T2_ROWS, 2 * D)
    out2 = _bond_encode(edge_attr[:, 0], edge_attr[:, 1], edge_attr[:, 2], t2)
    return _repack(out2).T
